# Initial kernel scaffold; baseline (speedup 1.0000x reference)
import jax
import jax.numpy as jnp
from jax.experimental import pallas as pl


def kernel(adj, x_features, W1, b1, W_mu, b_mu, W_lv, b_lv):
    raise NotImplementedError("write your pallas kernel here")


# speedup vs baseline: 12.7870x; 12.7870x over previous
import functools

import jax
import jax.numpy as jnp
from jax import lax
from jax.experimental import pallas as pl
from jax.experimental.pallas import tpu as pltpu
from jax.experimental.pallas import tpu_sc as plsc

N = 10000
E = 320000
IN_DIM = 128
HID = 128
LAT = 64
HH = HID // 2

NW = 32
K = 80
EPAD = NW * K * 128
ROWS = EPAD // 128
KT = ROWS // 16
NPAD = 10112
RPT = NPAD // 16

_MESH = plsc.VectorSubcoreMesh(core_axis_name="c", subcore_axis_name="s")
_SC_PARAMS = pltpu.CompilerParams(use_tc_tiling_on_sc=False)


@functools.partial(
    pl.kernel,
    mesh=_MESH,
    out_type=jax.ShapeDtypeStruct((2, NPAD, 16), jnp.float32),
    scratch_types=[
        pltpu.VMEM((K, 128), jnp.int32),
        pltpu.VMEM((128, 16), jnp.float32),
        pltpu.VMEM((RPT, 16), jnp.float32),
        pltpu.VMEM_SHARED((NPAD, 16), jnp.float32),
    ],
    compiler_params=_SC_PARAMS,
)
def _deg_sc(dst_hbm, ones_hbm, zer_hbm, out_hbm, dst_v, ones_v, zer_v, acc_sp):
    c = lax.axis_index("c")
    s = lax.axis_index("s")
    w = s * 2 + c
    pltpu.sync_copy(zer_hbm, zer_v)
    pltpu.sync_copy(zer_v, acc_sp.at[pl.ds(s * RPT, RPT)])
    pltpu.sync_copy(ones_hbm, ones_v)
    pltpu.sync_copy(dst_hbm.at[pl.ds(w * K, K)], dst_v)
    plsc.subcore_barrier()

    def body(j, _):
        pltpu.sync_copy(ones_v, acc_sp.at[dst_v.at[j]], add=True)
        return ()

    lax.fori_loop(0, K, body, ())
    plsc.subcore_barrier()
    pltpu.sync_copy(acc_sp.at[pl.ds(s * RPT, RPT)], out_hbm.at[c, pl.ds(s * RPT, RPT)])


@functools.partial(
    pl.kernel,
    mesh=_MESH,
    out_type=jax.ShapeDtypeStruct((2, NPAD, HH), jnp.float32),
    scratch_types=[
        pltpu.VMEM((KT, 128), jnp.int32),
        pltpu.VMEM((KT, 128), jnp.int32),
        pltpu.VMEM((2, 128, HH), jnp.float32),
        pltpu.VMEM_SHARED((NPAD, HH), jnp.float32),
        pltpu.SemaphoreType.DMA,
        pltpu.SemaphoreType.DMA,
    ],
    compiler_params=_SC_PARAMS,
)
def _prop_sc(y_hbm, src_hbm, dst_hbm, zer_hbm, out_hbm,
             src_v, dst_v, rows_v, acc_sp, sem0, sem1):
    c = lax.axis_index("c")
    s = lax.axis_index("s")
    pltpu.sync_copy(zer_hbm, rows_v.at[0])
    for r in range(4):
        pltpu.sync_copy(rows_v.at[0], acc_sp.at[pl.ds(s * RPT + r * 128, 128)])
    pltpu.sync_copy(rows_v.at[0, pl.ds(0, RPT - 512)],
                    acc_sp.at[pl.ds(s * RPT + 512, RPT - 512)])
    pltpu.sync_copy(src_hbm.at[pl.ds(c * ROWS + s * KT, KT)], src_v)
    pltpu.sync_copy(dst_hbm.at[pl.ds(s * KT, KT)], dst_v)
    plsc.subcore_barrier()

    pltpu.make_async_copy(y_hbm.at[src_v.at[0]], rows_v.at[0], sem0).start()

    def pair(i, _):
        j = i * 2
        pltpu.make_async_copy(y_hbm.at[src_v.at[j + 1]], rows_v.at[1], sem1).start()
        pltpu.make_async_copy(y_hbm.at[src_v.at[j]], rows_v.at[0], sem0).wait()
        pltpu.sync_copy(rows_v.at[0], acc_sp.at[dst_v.at[j]], add=True)

        @pl.when(j + 2 < KT)
        def _():
            pltpu.make_async_copy(y_hbm.at[src_v.at[j + 2]], rows_v.at[0], sem0).start()

        pltpu.make_async_copy(y_hbm.at[src_v.at[j + 1]], rows_v.at[1], sem1).wait()
        pltpu.sync_copy(rows_v.at[1], acc_sp.at[dst_v.at[j + 1]], add=True)
        return ()

    lax.fori_loop(0, KT // 2, pair, ())
    plsc.subcore_barrier()
    pltpu.sync_copy(acc_sp.at[pl.ds(s * RPT, RPT)], out_hbm.at[c, pl.ds(s * RPT, RPT)])


_BM = 1000


def _tc1_body(x_ref, w1_ref, dp_ref, y1_ref, dis_ref):
    deg = dp_ref[0, :, 0:1] + dp_ref[1, :, 0:1] + 1.0
    dis = lax.rsqrt(deg)
    xw = jnp.dot(x_ref[...], w1_ref[...], preferred_element_type=jnp.float32)
    y1_ref[0] = dis * xw[:, :HH]
    y1_ref[1] = dis * xw[:, HH:]
    dis_ref[...] = dis


def _tc2_body(s1_ref, y1_ref, dis_ref, b1_ref, yh_ref):
    dis = dis_ref[...]
    for c in range(2):
        h = jnp.maximum(dis * (s1_ref[c] + y1_ref[c]) + b1_ref[c], 0.0)
        yh_ref[c] = dis * h


def _tc3_body(s2_ref, yh_ref, dis_ref, wmu_ref, bmu_ref, wlv_ref, blv_ref,
              eps_ref, mu_ref, lv_ref, z_ref):
    dis = dis_ref[...]
    qa = dis * (s2_ref[0] + yh_ref[0])
    qb = dis * (s2_ref[1] + yh_ref[1])
    mu = (jnp.dot(qa, wmu_ref[0], preferred_element_type=jnp.float32)
          + jnp.dot(qb, wmu_ref[1], preferred_element_type=jnp.float32)
          + bmu_ref[...])
    lv = (jnp.dot(qa, wlv_ref[0], preferred_element_type=jnp.float32)
          + jnp.dot(qb, wlv_ref[1], preferred_element_type=jnp.float32)
          + blv_ref[...])
    mu_ref[...] = mu
    lv_ref[...] = lv
    z_ref[...] = eps_ref[...] * jnp.exp(0.5 * lv) + mu


_BD = 400


def _dec_body(zb_ref, z_ref, a_ref):
    prod = lax.dot_general(zb_ref[...], z_ref[...], (((1,), (1,)), ((), ())),
                           preferred_element_type=jnp.float32)
    a_ref[...] = jax.nn.sigmoid(prod)


def kernel(adj, x_features, W1, b1, W_mu, b_mu, W_lv, b_lv):
    f32 = jnp.float32
    src = adj[0].astype(jnp.int32)
    dst = adj[1].astype(jnp.int32)
    srcp = jnp.concatenate([src, jnp.zeros((EPAD - E,), jnp.int32)]).reshape(ROWS, 128)
    dstp = jnp.concatenate([dst, jnp.full((EPAD - E,), N, jnp.int32)]).reshape(ROWS, 128)
    srcc = jnp.concatenate([srcp, srcp + N])
    ones16 = jnp.ones((128, 16), f32)
    zer16 = jnp.zeros((RPT, 16), f32)
    zer64 = jnp.zeros((128, HH), f32)
    eps = jax.random.normal(jax.random.key(42), (N, LAT), dtype=f32)

    deg_p = _deg_sc(dstp, ones16, zer16)

    y1, dis = pl.pallas_call(
        _tc1_body,
        grid=(N // _BM,),
        in_specs=[
            pl.BlockSpec((_BM, IN_DIM), lambda i: (i, 0)),
            pl.BlockSpec((IN_DIM, HID), lambda i: (0, 0)),
            pl.BlockSpec((2, _BM, 16), lambda i: (0, i, 0)),
        ],
        out_specs=[
            pl.BlockSpec((2, _BM, HH), lambda i: (0, i, 0)),
            pl.BlockSpec((_BM, 1), lambda i: (i, 0)),
        ],
        out_shape=[
            jax.ShapeDtypeStruct((2, N, HH), f32),
            jax.ShapeDtypeStruct((N, 1), f32),
        ],
    )(x_features, W1, deg_p)

    s1 = _prop_sc(y1.reshape(2 * N, HH), srcc, dstp, zer64)

    yh = pl.pallas_call(
        _tc2_body,
        grid=(N // _BM,),
        in_specs=[
            pl.BlockSpec((2, _BM, HH), lambda i: (0, i, 0)),
            pl.BlockSpec((2, _BM, HH), lambda i: (0, i, 0)),
            pl.BlockSpec((_BM, 1), lambda i: (i, 0)),
            pl.BlockSpec((2, 1, HH), lambda i: (0, 0, 0)),
        ],
        out_specs=pl.BlockSpec((2, _BM, HH), lambda i: (0, i, 0)),
        out_shape=jax.ShapeDtypeStruct((2, N, HH), f32),
    )(s1, y1, dis, b1.reshape(2, 1, HH))

    s2 = _prop_sc(yh.reshape(2 * N, HH), srcc, dstp, zer64)

    mu, lv, z = pl.pallas_call(
        _tc3_body,
        grid=(N // _BM,),
        in_specs=[
            pl.BlockSpec((2, _BM, HH), lambda i: (0, i, 0)),
            pl.BlockSpec((2, _BM, HH), lambda i: (0, i, 0)),
            pl.BlockSpec((_BM, 1), lambda i: (i, 0)),
            pl.BlockSpec((2, HH, LAT), lambda i: (0, 0, 0)),
            pl.BlockSpec((1, LAT), lambda i: (0, 0)),
            pl.BlockSpec((2, HH, LAT), lambda i: (0, 0, 0)),
            pl.BlockSpec((1, LAT), lambda i: (0, 0)),
            pl.BlockSpec((_BM, LAT), lambda i: (i, 0)),
        ],
        out_specs=[
            pl.BlockSpec((_BM, LAT), lambda i: (i, 0)),
            pl.BlockSpec((_BM, LAT), lambda i: (i, 0)),
            pl.BlockSpec((_BM, LAT), lambda i: (i, 0)),
        ],
        out_shape=[
            jax.ShapeDtypeStruct((N, LAT), f32),
            jax.ShapeDtypeStruct((N, LAT), f32),
            jax.ShapeDtypeStruct((N, LAT), f32),
        ],
    )(s2, yh, dis, W_mu.reshape(2, HH, LAT), b_mu.reshape(1, LAT),
      W_lv.reshape(2, HH, LAT), b_lv.reshape(1, LAT), eps)

    a_pred = pl.pallas_call(
        _dec_body,
        grid=(N // _BD,),
        in_specs=[
            pl.BlockSpec((_BD, LAT), lambda i: (i, 0)),
            pl.BlockSpec((N, LAT), lambda i: (0, 0)),
        ],
        out_specs=pl.BlockSpec((_BD, N), lambda i: (i, 0)),
        out_shape=jax.ShapeDtypeStruct((N, N), f32),
    )(z, z)

    return (a_pred, mu, lv, z)


# speedup vs baseline: 13.2542x; 1.0365x over previous
import functools

import jax
import jax.numpy as jnp
from jax import lax
from jax.experimental import pallas as pl
from jax.experimental.pallas import tpu as pltpu
from jax.experimental.pallas import tpu_sc as plsc

N = 10000
E = 320000
IN_DIM = 128
HID = 128
LAT = 64
HH = HID // 2

NW = 32
K = 80
EPAD = NW * K * 128
ROWS = EPAD // 128
KT = ROWS // 16
NPAD = 10112
RPT = NPAD // 16

_MESH = plsc.VectorSubcoreMesh(core_axis_name="c", subcore_axis_name="s")
_SC_PARAMS = pltpu.CompilerParams(use_tc_tiling_on_sc=False)


@functools.partial(
    pl.kernel,
    mesh=_MESH,
    out_type=jax.ShapeDtypeStruct((2, NPAD, 16), jnp.float32),
    scratch_types=[
        pltpu.VMEM((K, 128), jnp.int32),
        pltpu.VMEM((128, 16), jnp.float32),
        pltpu.VMEM((RPT, 16), jnp.float32),
        pltpu.VMEM_SHARED((NPAD, 16), jnp.float32),
    ],
    compiler_params=_SC_PARAMS,
)
def _deg_sc(dst_hbm, ones_hbm, zer_hbm, out_hbm, dst_v, ones_v, zer_v, acc_sp):
    c = lax.axis_index("c")
    s = lax.axis_index("s")
    w = s * 2 + c
    pltpu.sync_copy(zer_hbm, zer_v)
    pltpu.sync_copy(zer_v, acc_sp.at[pl.ds(s * RPT, RPT)])
    pltpu.sync_copy(ones_hbm, ones_v)
    pltpu.sync_copy(dst_hbm.at[pl.ds(w * K, K)], dst_v)
    plsc.subcore_barrier()

    def body(j, _):
        pltpu.sync_copy(ones_v, acc_sp.at[dst_v.at[j]], add=True)
        return ()

    lax.fori_loop(0, K, body, ())
    plsc.subcore_barrier()
    pltpu.sync_copy(acc_sp.at[pl.ds(s * RPT, RPT)], out_hbm.at[c, pl.ds(s * RPT, RPT)])


@functools.partial(
    pl.kernel,
    mesh=_MESH,
    out_type=jax.ShapeDtypeStruct((2, NPAD, HH), jnp.float32),
    scratch_types=[
        pltpu.VMEM((KT, 128), jnp.int32),
        pltpu.VMEM((KT, 128), jnp.int32),
        pltpu.VMEM((5, 128, HH), jnp.float32),
        pltpu.VMEM_SHARED((NPAD, HH), jnp.float32),
        [pltpu.SemaphoreType.DMA] * 5,
        [pltpu.SemaphoreType.DMA] * 5,
    ],
    compiler_params=_SC_PARAMS,
)
def _prop_sc(y_hbm, src_hbm, dst_hbm, zer_hbm, out_hbm,
             src_v, dst_v, rows_v, acc_sp, sem_g, sem_s):
    c = lax.axis_index("c")
    s = lax.axis_index("s")
    pltpu.sync_copy(zer_hbm, rows_v.at[0])
    for r in range(4):
        pltpu.sync_copy(rows_v.at[0], acc_sp.at[pl.ds(s * RPT + r * 128, 128)])
    pltpu.sync_copy(rows_v.at[0, pl.ds(0, RPT - 512)],
                    acc_sp.at[pl.ds(s * RPT + 512, RPT - 512)])
    pltpu.sync_copy(src_hbm.at[pl.ds(c * ROWS + s * KT, KT)], src_v)
    pltpu.sync_copy(dst_hbm.at[pl.ds(s * KT, KT)], dst_v)
    plsc.subcore_barrier()

    def gather(j, b):
        pltpu.make_async_copy(y_hbm.at[src_v.at[j]], rows_v.at[b], sem_g[b]).start()

    def scatter_start(j, b):
        pltpu.async_copy(rows_v.at[b], acc_sp.at[dst_v.at[j]], sem_s[b], add=True)

    def scatter_wait(j, b):
        pltpu.make_async_copy(rows_v.at[b], acc_sp.at[dst_v.at[j]], sem_s[b]).wait()

    gather(0, 0)
    gather(1, 1)

    def ring(i, _):
        for b in range(5):
            j = i * 5 + b
            pltpu.make_async_copy(y_hbm.at[src_v.at[j]], rows_v.at[b], sem_g[b]).wait()
            scatter_start(j, b)
            b2 = (b + 2) % 5

            @pl.when(j >= 3)
            def _():
                scatter_wait(j - 3, b2)

            @pl.when(j + 2 < KT)
            def _():
                gather(j + 2, b2)
        return ()

    lax.fori_loop(0, KT // 5, ring, ())
    for j in range(KT - 3, KT):
        scatter_wait(j, j % 5)
    plsc.subcore_barrier()
    pltpu.sync_copy(acc_sp.at[pl.ds(s * RPT, RPT)], out_hbm.at[c, pl.ds(s * RPT, RPT)])


_BM = 1000


def _tc1_body(x_ref, w1_ref, dp_ref, y1_ref, dis_ref):
    deg = dp_ref[0, :, 0:1] + dp_ref[1, :, 0:1] + 1.0
    dis = lax.rsqrt(deg)
    xw = jnp.dot(x_ref[...], w1_ref[...], preferred_element_type=jnp.float32)
    y1_ref[0] = dis * xw[:, :HH]
    y1_ref[1] = dis * xw[:, HH:]
    dis_ref[...] = dis


def _tc2_body(s1_ref, y1_ref, dis_ref, b1_ref, yh_ref):
    dis = dis_ref[...]
    for c in range(2):
        h = jnp.maximum(dis * (s1_ref[c] + y1_ref[c]) + b1_ref[c], 0.0)
        yh_ref[c] = dis * h


def _tc3_body(s2_ref, yh_ref, dis_ref, wmu_ref, bmu_ref, wlv_ref, blv_ref,
              eps_ref, mu_ref, lv_ref, z_ref):
    dis = dis_ref[...]
    qa = dis * (s2_ref[0] + yh_ref[0])
    qb = dis * (s2_ref[1] + yh_ref[1])
    mu = (jnp.dot(qa, wmu_ref[0], preferred_element_type=jnp.float32)
          + jnp.dot(qb, wmu_ref[1], preferred_element_type=jnp.float32)
          + bmu_ref[...])
    lv = (jnp.dot(qa, wlv_ref[0], preferred_element_type=jnp.float32)
          + jnp.dot(qb, wlv_ref[1], preferred_element_type=jnp.float32)
          + blv_ref[...])
    mu_ref[...] = mu
    lv_ref[...] = lv
    z_ref[...] = eps_ref[...] * jnp.exp(0.5 * lv) + mu


_BD = 400


def _dec_body(zb_ref, z_ref, a_ref):
    prod = lax.dot_general(zb_ref[...], z_ref[...], (((1,), (1,)), ((), ())),
                           preferred_element_type=jnp.float32)
    a_ref[...] = jax.nn.sigmoid(prod)


def kernel(adj, x_features, W1, b1, W_mu, b_mu, W_lv, b_lv):
    f32 = jnp.float32
    src = adj[0].astype(jnp.int32)
    dst = adj[1].astype(jnp.int32)
    srcp = jnp.concatenate([src, jnp.zeros((EPAD - E,), jnp.int32)]).reshape(ROWS, 128)
    dstp = jnp.concatenate([dst, jnp.full((EPAD - E,), N, jnp.int32)]).reshape(ROWS, 128)
    srcc = jnp.concatenate([srcp, srcp + N])
    ones16 = jnp.ones((128, 16), f32)
    zer16 = jnp.zeros((RPT, 16), f32)
    zer64 = jnp.zeros((128, HH), f32)
    eps = jax.random.normal(jax.random.key(42), (N, LAT), dtype=f32)

    deg_p = _deg_sc(dstp, ones16, zer16)

    y1, dis = pl.pallas_call(
        _tc1_body,
        grid=(N // _BM,),
        in_specs=[
            pl.BlockSpec((_BM, IN_DIM), lambda i: (i, 0)),
            pl.BlockSpec((IN_DIM, HID), lambda i: (0, 0)),
            pl.BlockSpec((2, _BM, 16), lambda i: (0, i, 0)),
        ],
        out_specs=[
            pl.BlockSpec((2, _BM, HH), lambda i: (0, i, 0)),
            pl.BlockSpec((_BM, 1), lambda i: (i, 0)),
        ],
        out_shape=[
            jax.ShapeDtypeStruct((2, N, HH), f32),
            jax.ShapeDtypeStruct((N, 1), f32),
        ],
    )(x_features, W1, deg_p)

    s1 = _prop_sc(y1.reshape(2 * N, HH), srcc, dstp, zer64)

    yh = pl.pallas_call(
        _tc2_body,
        grid=(N // _BM,),
        in_specs=[
            pl.BlockSpec((2, _BM, HH), lambda i: (0, i, 0)),
            pl.BlockSpec((2, _BM, HH), lambda i: (0, i, 0)),
            pl.BlockSpec((_BM, 1), lambda i: (i, 0)),
            pl.BlockSpec((2, 1, HH), lambda i: (0, 0, 0)),
        ],
        out_specs=pl.BlockSpec((2, _BM, HH), lambda i: (0, i, 0)),
        out_shape=jax.ShapeDtypeStruct((2, N, HH), f32),
    )(s1, y1, dis, b1.reshape(2, 1, HH))

    s2 = _prop_sc(yh.reshape(2 * N, HH), srcc, dstp, zer64)

    mu, lv, z = pl.pallas_call(
        _tc3_body,
        grid=(N // _BM,),
        in_specs=[
            pl.BlockSpec((2, _BM, HH), lambda i: (0, i, 0)),
            pl.BlockSpec((2, _BM, HH), lambda i: (0, i, 0)),
            pl.BlockSpec((_BM, 1), lambda i: (i, 0)),
            pl.BlockSpec((2, HH, LAT), lambda i: (0, 0, 0)),
            pl.BlockSpec((1, LAT), lambda i: (0, 0)),
            pl.BlockSpec((2, HH, LAT), lambda i: (0, 0, 0)),
            pl.BlockSpec((1, LAT), lambda i: (0, 0)),
            pl.BlockSpec((_BM, LAT), lambda i: (i, 0)),
        ],
        out_specs=[
            pl.BlockSpec((_BM, LAT), lambda i: (i, 0)),
            pl.BlockSpec((_BM, LAT), lambda i: (i, 0)),
            pl.BlockSpec((_BM, LAT), lambda i: (i, 0)),
        ],
        out_shape=[
            jax.ShapeDtypeStruct((N, LAT), f32),
            jax.ShapeDtypeStruct((N, LAT), f32),
            jax.ShapeDtypeStruct((N, LAT), f32),
        ],
    )(s2, yh, dis, W_mu.reshape(2, HH, LAT), b_mu.reshape(1, LAT),
      W_lv.reshape(2, HH, LAT), b_lv.reshape(1, LAT), eps)

    a_pred = pl.pallas_call(
        _dec_body,
        grid=(N // _BD,),
        in_specs=[
            pl.BlockSpec((_BD, LAT), lambda i: (i, 0)),
            pl.BlockSpec((N, LAT), lambda i: (0, 0)),
        ],
        out_specs=pl.BlockSpec((_BD, N), lambda i: (i, 0)),
        out_shape=jax.ShapeDtypeStruct((N, N), f32),
    )(z, z)

    return (a_pred, mu, lv, z)
